# Initial kernel scaffold; baseline (speedup 1.0000x reference)
import jax
import jax.numpy as jnp
from jax.experimental import pallas as pl


def kernel(state, edge_index, conv_W, conv_b, lin1_W, lin1_b, lin2_W, lin2_b, lin3_W, lin3_b):
    raise NotImplementedError("write your pallas kernel here")


# speedup vs baseline: 20.6773x; 20.6773x over previous
import functools

import jax
import jax.numpy as jnp
from jax import lax
from jax.experimental import pallas as pl
from jax.experimental.pallas import tpu as pltpu
from jax.experimental.pallas import tpu_sc as plsc

_N = 10000
_E = 320000
_CH = 128
_NC = 2
_NS = 16
_NW = _NC * _NS
_EPW = _E // _NW
_K = 80
_NCHUNK = _EPW // _K
_NP = 10240
_RPT = _NP // _NS
_HW = 8


def _sc_degree_body(col_hbm, ones_hbm, zeros_hbm, out_hbm, col_v, ones_v, hist):
    core = lax.axis_index("c")
    sid = lax.axis_index("s")
    wid = sid * _NC + core
    pltpu.sync_copy(col_hbm.at[wid], col_v)
    pltpu.sync_copy(ones_hbm, ones_v)
    pltpu.sync_copy(zeros_hbm.at[pl.ds(sid * _RPT, _RPT)],
                    hist.at[pl.ds(sid * _RPT, _RPT)])
    plsc.subcore_barrier()

    def chunk(j, carry):
        pltpu.sync_copy(ones_v, hist.at[col_v.at[j]], add=True)
        return carry

    lax.fori_loop(0, _NCHUNK, chunk, 0)
    plsc.subcore_barrier()
    pltpu.sync_copy(hist.at[pl.ds(sid * _RPT, _RPT)],
                    out_hbm.at[core, pl.ds(sid * _RPT, _RPT)])


def _sc_aggregate_body(xs_hbm, row_hbm, col_hbm, zeros_hbm, out_hbm,
                       row_v, col_v, rows_v, acc, sem):
    core = lax.axis_index("c")
    sid = lax.axis_index("s")
    wid = sid * _NC + core
    pltpu.sync_copy(row_hbm.at[wid], row_v)
    pltpu.sync_copy(col_hbm.at[wid], col_v)
    pltpu.sync_copy(zeros_hbm.at[pl.ds(sid * _RPT, _RPT)],
                    acc.at[pl.ds(sid * _RPT, _RPT)])
    plsc.subcore_barrier()

    def chunk(j, carry):
        pltpu.async_copy(xs_hbm.at[row_v.at[j]], rows_v, sem).wait()
        pltpu.sync_copy(rows_v, acc.at[col_v.at[j]], add=True)
        return carry

    lax.fori_loop(0, _NCHUNK, chunk, 0)
    plsc.subcore_barrier()
    pltpu.sync_copy(acc.at[pl.ds(sid * _RPT, _RPT)],
                    out_hbm.at[core, pl.ds(sid * _RPT, _RPT)])


@jax.jit
def _sc_degree(col32, ones, zeros8):
    mesh = plsc.VectorSubcoreMesh(core_axis_name="c", subcore_axis_name="s")
    return pl.kernel(
        _sc_degree_body,
        out_type=jax.ShapeDtypeStruct((_NC, _NP, _HW), jnp.float32),
        mesh=mesh,
        scratch_types=[
            pltpu.VMEM((_NCHUNK, _K), jnp.int32),
            pltpu.VMEM((_K, _HW), jnp.float32),
            pltpu.VMEM_SHARED((_NP, _HW), jnp.float32),
        ],
    )(col32, ones, zeros8)


@jax.jit
def _sc_aggregate(xs, row32, col32, zeros):
    mesh = plsc.VectorSubcoreMesh(core_axis_name="c", subcore_axis_name="s")
    return pl.kernel(
        _sc_aggregate_body,
        out_type=jax.ShapeDtypeStruct((_NC, _NP, _CH), jnp.float32),
        mesh=mesh,
        scratch_types=[
            pltpu.VMEM((_NCHUNK, _K), jnp.int32),
            pltpu.VMEM((_NCHUNK, _K), jnp.int32),
            pltpu.VMEM((_K, _CH), jnp.float32),
            pltpu.VMEM_SHARED((_NP, _CH), jnp.float32),
            pltpu.SemaphoreType.DMA,
        ],
    )(xs, row32, col32, zeros)


def _tc_xs_body(state_ref, w_ref, h0_ref, h1_ref, xs_ref, dis_ref):
    xw = jnp.dot(state_ref[...], w_ref[...], preferred_element_type=jnp.float32)
    deg = h0_ref[...] + h1_ref[...] + 1.0
    dis = lax.rsqrt(deg)
    xs_ref[...] = dis * xw
    dis_ref[...] = dis


@jax.jit
def _tc_xs(state, conv_W, h0, h1):
    grid = 25
    rb = _N // grid
    return pl.pallas_call(
        _tc_xs_body,
        grid=(grid,),
        in_specs=[
            pl.BlockSpec((rb, _CH), lambda i: (i, 0)),
            pl.BlockSpec((_CH, _CH), lambda i: (0, 0)),
            pl.BlockSpec((rb, 1), lambda i: (i, 0)),
            pl.BlockSpec((rb, 1), lambda i: (i, 0)),
        ],
        out_specs=[
            pl.BlockSpec((rb, _CH), lambda i: (i, 0)),
            pl.BlockSpec((rb, 1), lambda i: (i, 0)),
        ],
        out_shape=[
            jax.ShapeDtypeStruct((_N, _CH), jnp.float32),
            jax.ShapeDtypeStruct((_N, 1), jnp.float32),
        ],
    )(state, conv_W, h0, h1)


def _tc_final_body(a0_ref, a1_ref, xs_ref, dis_ref, state_ref, cb_ref, s_ref,
                   w1_ref, b1_ref, w2_ref, b2_ref, w3_ref, b3_ref, out_ref):
    conv = dis_ref[...] * (a0_ref[...] + a1_ref[...] + xs_ref[...]) + cb_ref[...]
    h = jnp.maximum(conv, 0.0) + state_ref[...]
    g = jnp.dot(s_ref[...], h, preferred_element_type=jnp.float32)
    z = jnp.maximum(jnp.dot(g, w1_ref[...], preferred_element_type=jnp.float32)
                    + b1_ref[...], 0.0)
    z = jnp.maximum(jnp.dot(z, w2_ref[...], preferred_element_type=jnp.float32)
                    + b2_ref[...], 0.0)
    y = jnp.dot(z, w3_ref[...], preferred_element_type=jnp.float32) + b3_ref[...]
    out_ref[...] = jnp.broadcast_to(y[None], out_ref.shape)


@jax.jit
def _tc_final(a0, a1, xs, dis, state, conv_b, sel,
              lin1_W, lin1_b, lin2_W, lin2_b, lin3_W, lin3_b):
    grid = 25
    rb = _N // grid
    gb = rb // 16
    full = lambda shape: pl.BlockSpec(shape, lambda i: tuple(0 for _ in shape))
    return pl.pallas_call(
        _tc_final_body,
        grid=(grid,),
        in_specs=[
            pl.BlockSpec((rb, _CH), lambda i: (i, 0)),
            pl.BlockSpec((rb, _CH), lambda i: (i, 0)),
            pl.BlockSpec((rb, _CH), lambda i: (i, 0)),
            pl.BlockSpec((rb, 1), lambda i: (i, 0)),
            pl.BlockSpec((rb, _CH), lambda i: (i, 0)),
            full((1, _CH)),
            full((gb, rb)),
            full((_CH, 64)),
            full((1, 64)),
            full((64, 64)),
            full((1, 64)),
            full((64, 1)),
            full((1, 1)),
        ],
        out_specs=pl.BlockSpec((1, gb, _CH), lambda i: (i, 0, 0)),
        out_shape=jax.ShapeDtypeStruct((grid, gb, _CH), jnp.float32),
    )(a0, a1, xs, dis, state, conv_b, sel,
      lin1_W, lin1_b, lin2_W, lin2_b, lin3_W, lin3_b)


def kernel(state, edge_index, conv_W, conv_b, lin1_W, lin1_b, lin2_W, lin2_b,
           lin3_W, lin3_b):
    row32 = edge_index[0].astype(jnp.int32).reshape(_NW, _NCHUNK, _K)
    col32 = edge_index[1].astype(jnp.int32).reshape(_NW, _NCHUNK, _K)
    ones = jnp.ones((_K, _HW), jnp.float32)
    zeros8 = jnp.zeros((_NP, _HW), jnp.float32)
    zeros = jnp.zeros((_NP, _CH), jnp.float32)

    hist = _sc_degree(col32, ones, zeros8)
    h0 = hist[0, :_N, 0:1]
    h1 = hist[1, :_N, 0:1]
    xs, dis = _tc_xs(state, conv_W, h0, h1)
    acc = _sc_aggregate(xs, row32, col32, zeros)

    sel = (jnp.arange(25)[:, None] == (jnp.arange(400) // 16)[None, :]
           ).astype(jnp.float32)
    out3d = _tc_final(acc[0, :_N], acc[1, :_N], xs, dis, state, conv_b.reshape(1, _CH),
                      sel, lin1_W, lin1_b.reshape(1, 64), lin2_W,
                      lin2_b.reshape(1, 64), lin3_W, lin3_b.reshape(1, 1))
    return out3d.reshape(_N // 16, _CH)[:, 0]
